# baseline (speedup 1.0000x reference)
import functools

import jax
import jax.numpy as jnp
from jax import lax
from jax.experimental import pallas as pl
from jax.experimental.pallas import tpu as pltpu
from jax.experimental.pallas import tpu_sc as plsc

_NC = 2
_NS = 16
_CHUNK = 80


def _deg_pad(N):
    unit = 128 * _NS
    return ((N + unit - 1) // unit) * unit


def _make_sc_aggregate(N, D, E, with_deg):
    NW = _NC * _NS
    ep = E // NW
    nch = ep // _CHUNK
    rp = (N // (8 * _NS)) * 8
    tail = N - _NS * rp
    npad = _deg_pad(N)
    dp = npad // _NS

    mesh = plsc.VectorSubcoreMesh(core_axis_name="c", subcore_axis_name="s")

    out_type = jax.ShapeDtypeStruct((_NC, N, D), jnp.float32)
    scratch = [
        [pltpu.VMEM((_CHUNK,), jnp.int32) for _ in range(8)],
        [pltpu.VMEM((_CHUNK,), jnp.int32) for _ in range(8)],
        [pltpu.VMEM((_CHUNK, D), jnp.float32) for _ in range(4)],
        pltpu.VMEM_SHARED((N, D), jnp.float32),
        [pltpu.SemaphoreType.DMA for _ in range(8)],
        [pltpu.SemaphoreType.DMA for _ in range(4)],
        [pltpu.SemaphoreType.DMA for _ in range(4)],
    ]
    if with_deg:
        out_type = [out_type, jax.ShapeDtypeStruct((_NC, npad), jnp.float32)]
        scratch += [
            pltpu.VMEM((_CHUNK,), jnp.float32),
            pltpu.VMEM((128,), jnp.float32),
            pltpu.VMEM_SHARED((npad,), jnp.float32),
        ]

    @functools.partial(pl.kernel, mesh=mesh, out_type=out_type,
                       scratch_types=scratch)
    def agg(t_hbm, edge_hbm, *refs):
        if with_deg:
            (out_hbm, deg_hbm, rowb, colb, msgs, agg_sh,
             isems, gsems, ssems, ones_v, zero_v, deg_sh) = refs
        else:
            out_hbm, rowb, colb, msgs, agg_sh, isems, gsems, ssems = refs
        cid = lax.axis_index("c")
        sid = lax.axis_index("s")
        wid = cid * _NS + sid
        e0 = wid * ep
        c0 = E + wid * ep

        def start_idx(j, ib):
            pltpu.async_copy(edge_hbm.at[pl.ds(e0 + j * _CHUNK, _CHUNK)],
                             rowb[ib], isems[ib])
            pltpu.async_copy(edge_hbm.at[pl.ds(c0 + j * _CHUNK, _CHUNK)],
                             colb[ib], isems[ib])

        def wait_idx(j, ib):
            pltpu.make_async_copy(edge_hbm.at[pl.ds(e0 + j * _CHUNK, _CHUNK)],
                                  rowb[ib], isems[ib]).wait()
            pltpu.make_async_copy(edge_hbm.at[pl.ds(c0 + j * _CHUNK, _CHUNK)],
                                  colb[ib], isems[ib]).wait()

        def start_gather(b, ib):
            pltpu.async_copy(t_hbm.at[rowb[ib]], msgs[b], gsems[b])

        def wait_gather(b, ib):
            pltpu.make_async_copy(t_hbm.at[rowb[ib]], msgs[b], gsems[b]).wait()

        def start_scatter(b, ib):
            pltpu.async_copy(msgs[b], agg_sh.at[colb[ib]], ssems[b], add=True)
            if with_deg:
                pltpu.async_copy(ones_v, deg_sh.at[colb[ib]], ssems[b],
                                 add=True)

        def drain_scatter(b, ib):
            pltpu.make_async_copy(msgs[b], agg_sh.at[colb[ib]], ssems[b]).wait()
            if with_deg:
                pltpu.make_async_copy(ones_v, deg_sh.at[colb[ib]],
                                      ssems[b]).wait()

        for k in range(7):
            start_idx(k, k)
        for k in range(3):
            wait_idx(k, k)
            start_gather(k, k)

        r0 = sid * rp
        pltpu.sync_copy(t_hbm.at[pl.ds(r0, rp)], agg_sh.at[pl.ds(r0, rp)])
        if tail:
            @pl.when(sid == 0)
            def _():
                pltpu.sync_copy(t_hbm.at[pl.ds(_NS * rp, tail)],
                                agg_sh.at[pl.ds(_NS * rp, tail)])
        if with_deg:
            for k in range(_CHUNK // 16):
                ones_v[pl.ds(16 * k, 16)] = jnp.full((16,), 1.0, jnp.float32)
            for k in range(8):
                zero_v[pl.ds(16 * k, 16)] = jnp.zeros((16,), jnp.float32)
            for m in range(dp // 128):
                pltpu.sync_copy(zero_v,
                                deg_sh.at[pl.ds(sid * dp + 128 * m, 128)])
        plsc.subcore_barrier()

        def body(i, carry):
            j0 = 8 * i
            for p in range(8):
                j = j0 + p
                b = p % 4

                @pl.when(j < nch)
                def _():
                    wait_gather(b, p)
                    if p == 0:
                        @pl.when(j >= 1)
                        def _():
                            drain_scatter((b + 3) % 4, (p + 7) % 8)
                    else:
                        drain_scatter((b + 3) % 4, (p + 7) % 8)

                    @pl.when(j + 3 < nch)
                    def _():
                        wait_idx(j + 3, (p + 3) % 8)
                        start_gather((b + 3) % 4, (p + 3) % 8)

                    @pl.when(j + 7 < nch)
                    def _():
                        start_idx(j + 7, (p + 7) % 8)
                    start_scatter(b, p)
            return carry

        lax.fori_loop(0, (nch + 7) // 8, body, 0)
        drain_scatter((nch - 1) % 4, (nch - 1) % 8)
        plsc.subcore_barrier()
        pltpu.sync_copy(agg_sh.at[pl.ds(r0, rp)],
                        out_hbm.at[cid, pl.ds(r0, rp)])
        if tail:
            @pl.when(sid == 0)
            def _():
                pltpu.sync_copy(agg_sh.at[pl.ds(_NS * rp, tail)],
                                out_hbm.at[cid, pl.ds(_NS * rp, tail)])
        if with_deg:
            pltpu.sync_copy(deg_sh.at[pl.ds(sid * dp, dp)],
                            deg_hbm.at[cid, pl.ds(sid * dp, dp)])

    return agg


_BR = 1024


def _tc_transform(p, t, dg, w, b, relu):
    N, D = p.shape[1], p.shape[2]

    def body(p_ref, t_ref, dg_ref, w_ref, b_ref, o_ref):
        s = p_ref[0] + p_ref[1] - t_ref[...]
        scale = (1.0 + dg_ref[0] + dg_ref[1])[:, None]
        o = lax.dot_general(
            s, w_ref[...] * 0.5, (((1,), (1,)), ((), ())),
            preferred_element_type=jnp.float32) + scale * (b_ref[...] * 0.5)
        if relu:
            o = jnp.maximum(o, 0.0)
        o_ref[...] = o

    return pl.pallas_call(
        body,
        grid=((N + _BR - 1) // _BR,),
        in_specs=[
            pl.BlockSpec((_NC, _BR, D), lambda i: (0, i, 0)),
            pl.BlockSpec((_BR, D), lambda i: (i, 0)),
            pl.BlockSpec((_NC, _BR), lambda i: (0, i)),
            pl.BlockSpec((D, D), lambda i: (0, 0)),
            pl.BlockSpec((1, D), lambda i: (0, 0)),
        ],
        out_specs=pl.BlockSpec((_BR, D), lambda i: (i, 0)),
        out_shape=jax.ShapeDtypeStruct((N, D), jnp.float32),
    )(p, t, dg, w, b)


def kernel(node_features, edge_index, weight0, bias0, weight1, bias1,
           hidden_dim):
    N, D = node_features.shape
    E = edge_index.shape[1]
    edges = edge_index.reshape(2 * E)

    sc_agg_deg = _make_sc_aggregate(N, D, E, True)
    sc_agg = _make_sc_aggregate(N, D, E, False)

    p1, deg = sc_agg_deg(node_features, edges)
    h = _tc_transform(p1, node_features, deg, weight0[0], bias0, True)
    p2 = sc_agg(h, edges)
    return _tc_transform(p2, h, deg, weight1[0], bias1, False)


# speedup vs baseline: 1.0014x; 1.0014x over previous
import functools

import jax
import jax.numpy as jnp
from jax import lax
from jax.experimental import pallas as pl
from jax.experimental.pallas import tpu as pltpu
from jax.experimental.pallas import tpu_sc as plsc

_NC = 2
_NS = 16
_CHUNK = 80


def _deg_pad(N):
    unit = 128 * _NS
    return ((N + unit - 1) // unit) * unit


def _make_sc_aggregate(N, D, E, with_deg):
    NW = _NC * _NS
    ep = E // NW
    nch = ep // _CHUNK
    rp = (N // (8 * _NS)) * 8
    tail = N - _NS * rp
    npad = _deg_pad(N)
    dp = npad // _NS

    mesh = plsc.VectorSubcoreMesh(core_axis_name="c", subcore_axis_name="s")

    out_type = jax.ShapeDtypeStruct((_NC, N, D), jnp.float32)
    scratch = [
        [pltpu.VMEM((_CHUNK,), jnp.int32) for _ in range(8)],
        [pltpu.VMEM((_CHUNK,), jnp.int32) for _ in range(8)],
        [pltpu.VMEM((_CHUNK, D), jnp.float32) for _ in range(4)],
        pltpu.VMEM_SHARED((N, D), jnp.float32),
        [pltpu.SemaphoreType.DMA for _ in range(8)],
        [pltpu.SemaphoreType.DMA for _ in range(4)],
        [pltpu.SemaphoreType.DMA for _ in range(4)],
    ]
    if with_deg:
        out_type = [out_type, jax.ShapeDtypeStruct((_NC, npad), jnp.float32)]
        scratch += [
            pltpu.VMEM((_CHUNK,), jnp.float32),
            pltpu.VMEM((128,), jnp.float32),
            pltpu.VMEM_SHARED((npad,), jnp.float32),
        ]

    @functools.partial(pl.kernel, mesh=mesh, out_type=out_type,
                       scratch_types=scratch)
    def agg(t_hbm, edge_hbm, *refs):
        if with_deg:
            (out_hbm, deg_hbm, rowb, colb, msgs, agg_sh,
             isems, gsems, ssems, ones_v, zero_v, deg_sh) = refs
        else:
            out_hbm, rowb, colb, msgs, agg_sh, isems, gsems, ssems = refs
        cid = lax.axis_index("c")
        sid = lax.axis_index("s")
        wid = cid * _NS + sid
        e0 = wid * ep
        c0 = E + wid * ep

        def start_idx(j, ib):
            pltpu.async_copy(edge_hbm.at[pl.ds(e0 + j * _CHUNK, _CHUNK)],
                             rowb[ib], isems[ib])
            pltpu.async_copy(edge_hbm.at[pl.ds(c0 + j * _CHUNK, _CHUNK)],
                             colb[ib], isems[ib])

        def wait_idx(j, ib):
            pltpu.make_async_copy(edge_hbm.at[pl.ds(e0 + j * _CHUNK, _CHUNK)],
                                  rowb[ib], isems[ib]).wait()
            pltpu.make_async_copy(edge_hbm.at[pl.ds(c0 + j * _CHUNK, _CHUNK)],
                                  colb[ib], isems[ib]).wait()

        _SPL = 48

        def start_gather(b, ib):
            pltpu.async_copy(t_hbm.at[rowb[ib].at[pl.ds(0, _SPL)]],
                             msgs[b].at[pl.ds(0, _SPL)], gsems[b])
            pltpu.async_copy(t_hbm.at[rowb[ib].at[pl.ds(_SPL, _CHUNK - _SPL)]],
                             msgs[b].at[pl.ds(_SPL, _CHUNK - _SPL)], gsems[b])

        def wait_gather(b, ib):
            pltpu.make_async_copy(t_hbm.at[rowb[ib].at[pl.ds(0, _SPL)]],
                                  msgs[b].at[pl.ds(0, _SPL)], gsems[b]).wait()
            pltpu.make_async_copy(t_hbm.at[rowb[ib].at[pl.ds(_SPL, _CHUNK - _SPL)]],
                                  msgs[b].at[pl.ds(_SPL, _CHUNK - _SPL)],
                                  gsems[b]).wait()

        def start_scatter(b, ib):
            pltpu.async_copy(msgs[b], agg_sh.at[colb[ib]], ssems[b], add=True)
            if with_deg:
                pltpu.async_copy(ones_v, deg_sh.at[colb[ib]], ssems[b],
                                 add=True)

        def drain_scatter(b, ib):
            pltpu.make_async_copy(msgs[b], agg_sh.at[colb[ib]], ssems[b]).wait()
            if with_deg:
                pltpu.make_async_copy(ones_v, deg_sh.at[colb[ib]],
                                      ssems[b]).wait()

        for k in range(7):
            start_idx(k, k)
        for k in range(3):
            wait_idx(k, k)
            start_gather(k, k)

        r0 = sid * rp
        pltpu.sync_copy(t_hbm.at[pl.ds(r0, rp)], agg_sh.at[pl.ds(r0, rp)])
        if tail:
            @pl.when(sid == 0)
            def _():
                pltpu.sync_copy(t_hbm.at[pl.ds(_NS * rp, tail)],
                                agg_sh.at[pl.ds(_NS * rp, tail)])
        if with_deg:
            for k in range(_CHUNK // 16):
                ones_v[pl.ds(16 * k, 16)] = jnp.full((16,), 1.0, jnp.float32)
            for k in range(8):
                zero_v[pl.ds(16 * k, 16)] = jnp.zeros((16,), jnp.float32)
            for m in range(dp // 128):
                pltpu.sync_copy(zero_v,
                                deg_sh.at[pl.ds(sid * dp + 128 * m, 128)])
        plsc.subcore_barrier()

        def body(i, carry):
            j0 = 8 * i
            for p in range(8):
                j = j0 + p
                b = p % 4

                @pl.when(j < nch)
                def _():
                    wait_gather(b, p)
                    if p == 0:
                        @pl.when(j >= 1)
                        def _():
                            drain_scatter((b + 3) % 4, (p + 7) % 8)
                    else:
                        drain_scatter((b + 3) % 4, (p + 7) % 8)

                    @pl.when(j + 3 < nch)
                    def _():
                        wait_idx(j + 3, (p + 3) % 8)
                        start_gather((b + 3) % 4, (p + 3) % 8)

                    @pl.when(j + 7 < nch)
                    def _():
                        start_idx(j + 7, (p + 7) % 8)
                    start_scatter(b, p)
            return carry

        lax.fori_loop(0, (nch + 7) // 8, body, 0)
        drain_scatter((nch - 1) % 4, (nch - 1) % 8)
        plsc.subcore_barrier()
        pltpu.sync_copy(agg_sh.at[pl.ds(r0, rp)],
                        out_hbm.at[cid, pl.ds(r0, rp)])
        if tail:
            @pl.when(sid == 0)
            def _():
                pltpu.sync_copy(agg_sh.at[pl.ds(_NS * rp, tail)],
                                out_hbm.at[cid, pl.ds(_NS * rp, tail)])
        if with_deg:
            pltpu.sync_copy(deg_sh.at[pl.ds(sid * dp, dp)],
                            deg_hbm.at[cid, pl.ds(sid * dp, dp)])

    return agg


_BR = 1024


def _tc_transform(p, t, dg, w, b, relu):
    N, D = p.shape[1], p.shape[2]

    def body(p_ref, t_ref, dg_ref, w_ref, b_ref, o_ref):
        s = p_ref[0] + p_ref[1] - t_ref[...]
        scale = (1.0 + dg_ref[0] + dg_ref[1])[:, None]
        o = lax.dot_general(
            s, w_ref[...] * 0.5, (((1,), (1,)), ((), ())),
            preferred_element_type=jnp.float32) + scale * (b_ref[...] * 0.5)
        if relu:
            o = jnp.maximum(o, 0.0)
        o_ref[...] = o

    return pl.pallas_call(
        body,
        grid=((N + _BR - 1) // _BR,),
        in_specs=[
            pl.BlockSpec((_NC, _BR, D), lambda i: (0, i, 0)),
            pl.BlockSpec((_BR, D), lambda i: (i, 0)),
            pl.BlockSpec((_NC, _BR), lambda i: (0, i)),
            pl.BlockSpec((D, D), lambda i: (0, 0)),
            pl.BlockSpec((1, D), lambda i: (0, 0)),
        ],
        out_specs=pl.BlockSpec((_BR, D), lambda i: (i, 0)),
        out_shape=jax.ShapeDtypeStruct((N, D), jnp.float32),
    )(p, t, dg, w, b)


def kernel(node_features, edge_index, weight0, bias0, weight1, bias1,
           hidden_dim):
    N, D = node_features.shape
    E = edge_index.shape[1]
    edges = edge_index.reshape(2 * E)

    sc_agg_deg = _make_sc_aggregate(N, D, E, True)
    sc_agg = _make_sc_aggregate(N, D, E, False)

    p1, deg = sc_agg_deg(node_features, edges)
    h = _tc_transform(p1, node_features, deg, weight0[0], bias0, True)
    p2 = sc_agg(h, edges)
    return _tc_transform(p2, h, deg, weight1[0], bias1, False)


# speedup vs baseline: 1.0016x; 1.0002x over previous
import functools

import jax
import jax.numpy as jnp
from jax import lax
from jax.experimental import pallas as pl
from jax.experimental.pallas import tpu as pltpu
from jax.experimental.pallas import tpu_sc as plsc

_NC = 2
_NS = 16
_CHUNK = 80


def _deg_pad(N):
    unit = 128 * _NS
    return ((N + unit - 1) // unit) * unit


def _make_sc_aggregate(N, D, E, with_deg):
    NW = _NC * _NS
    ep = E // NW
    nch = ep // _CHUNK
    rp = (N // (8 * _NS)) * 8
    tail = N - _NS * rp
    npad = _deg_pad(N)
    dp = npad // _NS

    mesh = plsc.VectorSubcoreMesh(core_axis_name="c", subcore_axis_name="s")

    out_type = jax.ShapeDtypeStruct((_NC, N, D), jnp.float32)
    scratch = [
        [pltpu.VMEM((_CHUNK,), jnp.int32) for _ in range(8)],
        [pltpu.VMEM((_CHUNK,), jnp.int32) for _ in range(8)],
        [pltpu.VMEM((_CHUNK, D), jnp.float32) for _ in range(4)],
        pltpu.VMEM_SHARED((N, D), jnp.float32),
        [pltpu.SemaphoreType.DMA for _ in range(8)],
        [pltpu.SemaphoreType.DMA for _ in range(4)],
        [pltpu.SemaphoreType.DMA for _ in range(4)],
    ]
    if with_deg:
        out_type = [out_type, jax.ShapeDtypeStruct((_NC, npad), jnp.float32)]
        scratch += [
            pltpu.VMEM((_CHUNK,), jnp.float32),
            pltpu.VMEM((128,), jnp.float32),
            pltpu.VMEM_SHARED((npad,), jnp.float32),
        ]

    @functools.partial(pl.kernel, mesh=mesh, out_type=out_type,
                       scratch_types=scratch)
    def agg(t_hbm, edge_hbm, *refs):
        if with_deg:
            (out_hbm, deg_hbm, rowb, colb, msgs, agg_sh,
             isems, gsems, ssems, ones_v, zero_v, deg_sh) = refs
        else:
            out_hbm, rowb, colb, msgs, agg_sh, isems, gsems, ssems = refs
        cid = lax.axis_index("c")
        sid = lax.axis_index("s")
        wid = cid * _NS + sid
        e0 = wid * ep
        c0 = E + wid * ep

        def start_idx(j, ib):
            pltpu.async_copy(edge_hbm.at[pl.ds(e0 + j * _CHUNK, _CHUNK)],
                             rowb[ib], isems[ib])
            pltpu.async_copy(edge_hbm.at[pl.ds(c0 + j * _CHUNK, _CHUNK)],
                             colb[ib], isems[ib])

        def wait_idx(j, ib):
            pltpu.make_async_copy(edge_hbm.at[pl.ds(e0 + j * _CHUNK, _CHUNK)],
                                  rowb[ib], isems[ib]).wait()
            pltpu.make_async_copy(edge_hbm.at[pl.ds(c0 + j * _CHUNK, _CHUNK)],
                                  colb[ib], isems[ib]).wait()

        def start_gather(b, ib):
            pltpu.async_copy(t_hbm.at[rowb[ib]], msgs[b], gsems[b])

        def wait_gather(b, ib):
            pltpu.make_async_copy(t_hbm.at[rowb[ib]], msgs[b], gsems[b]).wait()

        def start_scatter(b, ib):
            pltpu.async_copy(msgs[b], agg_sh.at[colb[ib]], ssems[b], add=True)
            if with_deg:
                pltpu.async_copy(ones_v, deg_sh.at[colb[ib]], ssems[b],
                                 add=True)

        def drain_scatter(b, ib):
            pltpu.make_async_copy(msgs[b], agg_sh.at[colb[ib]], ssems[b]).wait()
            if with_deg:
                pltpu.make_async_copy(ones_v, deg_sh.at[colb[ib]],
                                      ssems[b]).wait()

        for k in range(5):
            start_idx(k, k)
        for k in range(3):
            wait_idx(k, k)
            start_gather(k, k)

        r0 = sid * rp
        pltpu.sync_copy(t_hbm.at[pl.ds(r0, rp)], agg_sh.at[pl.ds(r0, rp)])
        if tail:
            @pl.when(sid == 0)
            def _():
                pltpu.sync_copy(t_hbm.at[pl.ds(_NS * rp, tail)],
                                agg_sh.at[pl.ds(_NS * rp, tail)])
        if with_deg:
            for k in range(_CHUNK // 16):
                ones_v[pl.ds(16 * k, 16)] = jnp.full((16,), 1.0, jnp.float32)
            for k in range(8):
                zero_v[pl.ds(16 * k, 16)] = jnp.zeros((16,), jnp.float32)
            for m in range(dp // 128):
                pltpu.sync_copy(zero_v,
                                deg_sh.at[pl.ds(sid * dp + 128 * m, 128)])
        plsc.subcore_barrier()

        def body(i, carry):
            j0 = 8 * i
            for p in range(8):
                j = j0 + p
                b = p % 4

                @pl.when(j < nch)
                def _():
                    wait_gather(b, p)
                    if p == 0:
                        @pl.when(j >= 1)
                        def _():
                            drain_scatter((b + 3) % 4, (p + 7) % 8)
                    else:
                        drain_scatter((b + 3) % 4, (p + 7) % 8)

                    @pl.when(j + 3 < nch)
                    def _():
                        wait_idx(j + 3, (p + 3) % 8)
                        start_gather((b + 3) % 4, (p + 3) % 8)

                    @pl.when(j + 5 < nch)
                    def _():
                        start_idx(j + 5, (p + 5) % 8)
                    start_scatter(b, p)
            return carry

        lax.fori_loop(0, (nch + 7) // 8, body, 0)
        drain_scatter((nch - 1) % 4, (nch - 1) % 8)
        plsc.subcore_barrier()
        pltpu.sync_copy(agg_sh.at[pl.ds(r0, rp)],
                        out_hbm.at[cid, pl.ds(r0, rp)])
        if tail:
            @pl.when(sid == 0)
            def _():
                pltpu.sync_copy(agg_sh.at[pl.ds(_NS * rp, tail)],
                                out_hbm.at[cid, pl.ds(_NS * rp, tail)])
        if with_deg:
            pltpu.sync_copy(deg_sh.at[pl.ds(sid * dp, dp)],
                            deg_hbm.at[cid, pl.ds(sid * dp, dp)])

    return agg


_BR = 1024


def _tc_transform(p, t, dg, w, b, relu):
    N, D = p.shape[1], p.shape[2]

    def body(p_ref, t_ref, dg_ref, w_ref, b_ref, o_ref):
        s = p_ref[0] + p_ref[1] - t_ref[...]
        scale = (1.0 + dg_ref[0] + dg_ref[1])[:, None]
        o = lax.dot_general(
            s, w_ref[...] * 0.5, (((1,), (1,)), ((), ())),
            preferred_element_type=jnp.float32) + scale * (b_ref[...] * 0.5)
        if relu:
            o = jnp.maximum(o, 0.0)
        o_ref[...] = o

    return pl.pallas_call(
        body,
        grid=((N + _BR - 1) // _BR,),
        in_specs=[
            pl.BlockSpec((_NC, _BR, D), lambda i: (0, i, 0)),
            pl.BlockSpec((_BR, D), lambda i: (i, 0)),
            pl.BlockSpec((_NC, _BR), lambda i: (0, i)),
            pl.BlockSpec((D, D), lambda i: (0, 0)),
            pl.BlockSpec((1, D), lambda i: (0, 0)),
        ],
        out_specs=pl.BlockSpec((_BR, D), lambda i: (i, 0)),
        out_shape=jax.ShapeDtypeStruct((N, D), jnp.float32),
    )(p, t, dg, w, b)


def kernel(node_features, edge_index, weight0, bias0, weight1, bias1,
           hidden_dim):
    N, D = node_features.shape
    E = edge_index.shape[1]
    edges = edge_index.reshape(2 * E)

    sc_agg_deg = _make_sc_aggregate(N, D, E, True)
    sc_agg = _make_sc_aggregate(N, D, E, False)

    p1, deg = sc_agg_deg(node_features, edges)
    h = _tc_transform(p1, node_features, deg, weight0[0], bias0, True)
    p2 = sc_agg(h, edges)
    return _tc_transform(p2, h, deg, weight1[0], bias1, False)
